# Initial kernel scaffold; baseline (speedup 1.0000x reference)
import jax
import jax.numpy as jnp
from jax.experimental import pallas as pl


def kernel(x, y, y_neg):
    raise NotImplementedError("write your pallas kernel here")


# speedup vs baseline: 12.8664x; 12.8664x over previous
import functools
import jax
import jax.numpy as jnp
from jax import lax
from jax.experimental import pallas as pl
from jax.experimental.pallas import tpu as pltpu
from jax.experimental.pallas import tpu_sc as plsc

B = 4096
C = 9605
L = 10
GROUP = 64
ALPHA = 0.3
ALPHA1 = 0.05
ALPHA3 = 5.0

NL = 16
NCHUNK = (C + NL - 1) // NL
NFULL = NCHUNK - 1
NTAIL = C - NFULL * NL
NW = 32
RPW = B // NW
NEG = float("-inf")
KTH = 10


def _sorted16(v, descending=False):
    k, _ = plsc.sort_key_val(v, v, descending=descending)
    return k


def _top4_merge(m1, m2, m3, m4):
    a = _sorted16(m1)
    b = _sorted16(m2, descending=True)
    h1 = _sorted16(jnp.maximum(a, b))
    c = _sorted16(m3)
    d = _sorted16(m4, descending=True)
    h2 = _sorted16(jnp.maximum(c, d), descending=True)
    hs = _sorted16(jnp.maximum(h1, h2))
    ii = lax.iota(jnp.int32, NL)
    return jnp.max(jnp.where(ii == NL - 1 - KTH, hs, NEG))


SLAB = 8
NSLAB = RPW // SLAB


def _sc_body(x_hbm, y_hbm, yn_hbm, out_hbm, xbuf, ybuf, ynbuf, outb):
    cid = lax.axis_index("c")
    sid = lax.axis_index("s")
    wid = sid * 2 + cid
    base = wid * RPW
    ii = lax.iota(jnp.int32, NL)
    zeros = jnp.zeros((NL,), jnp.float32)
    neginf = jnp.full((NL,), -jnp.inf, jnp.float32)
    tail_mask = ii < NTAIL

    def slab_body(sb, acc0):
        row0 = base + sb * SLAB
        pltpu.sync_copy(x_hbm.at[pl.ds(row0, SLAB), :], xbuf)
        pltpu.sync_copy(y_hbm.at[pl.ds(row0, SLAB), pl.ds(0, L * GROUP)], ybuf)
        pltpu.sync_copy(yn_hbm.at[pl.ds(row0, SLAB), pl.ds(0, L * GROUP)], ynbuf)
        return lax.fori_loop(0, SLAB, row_body, acc0)

    def load_tail(r, thr):
        idx_r = jnp.zeros((NL,), jnp.int32) + r
        idx_c = jnp.minimum(NFULL * NL + ii, C - 1)
        v = plsc.load_gather(xbuf, [idx_r, idx_c])
        return jnp.where(tail_mask, v, thr)

    def group_max10(buf, r):
        out = neginf
        for l in range(L):
            g0 = buf[r, pl.ds(l * GROUP, NL)]
            g1 = buf[r, pl.ds(l * GROUP + NL, NL)]
            g2 = buf[r, pl.ds(l * GROUP + 2 * NL, NL)]
            g3 = buf[r, pl.ds(l * GROUP + 3 * NL, NL)]
            s = jnp.max(jnp.maximum(jnp.maximum(g0, g1), jnp.maximum(g2, g3)))
            out = jnp.where(ii == l, s, out)
        return out

    def row_body(r, acc):
        def c_body(j, ms):
            m1, m2, m3, m4 = ms
            v = xbuf[r, pl.ds(j * NL, NL)]
            t = jnp.minimum(m1, v); m1 = jnp.maximum(m1, v)
            u = jnp.minimum(m2, t); m2 = jnp.maximum(m2, t)
            w = jnp.minimum(m3, u); m3 = jnp.maximum(m3, u)
            m4 = jnp.maximum(m4, w)
            return (m1, m2, m3, m4)

        m1, m2, m3, m4 = lax.fori_loop(
            0, NFULL, c_body, (neginf, neginf, neginf, neginf))
        vl = load_tail(r, neginf)
        t = jnp.minimum(m1, vl); m1 = jnp.maximum(m1, vl)
        u = jnp.minimum(m2, t); m2 = jnp.maximum(m2, t)
        w = jnp.minimum(m3, u); m3 = jnp.maximum(m3, u)
        m4 = jnp.maximum(m4, w)

        tk = _top4_merge(m1, m2, m3, m4)

        def v_body(j, cnt):
            v = xbuf[r, pl.ds(j * NL, NL)]
            return cnt + jnp.where(v > tk, 1.0, 0.0)

        cntv = lax.fori_loop(0, NFULL, v_body, zeros)
        cntv = cntv + jnp.where(load_tail(r, NEG) > tk, 1.0, 0.0)
        c_gt = jnp.sum(cntv)

        def fallback():
            def extract(_, st):
                thr, cnt, ans = st

                def p_body(j, mc):
                    mx, ct = mc
                    v = xbuf[r, pl.ds(j * NL, NL)]
                    below = jnp.where(v < thr, v, -jnp.inf)
                    mx = jnp.maximum(mx, below)
                    return (mx, ct)

                mx, _ = lax.fori_loop(0, NFULL, p_body, (neginf, zeros))
                vt = load_tail(r, thr)
                mx = jnp.maximum(mx, jnp.where(vt < thr, vt, -jnp.inf))
                cur = jnp.max(mx)

                def e_body(j, ct):
                    v = xbuf[r, pl.ds(j * NL, NL)]
                    return ct + jnp.where(v == cur, 1.0, 0.0)

                eqv = lax.fori_loop(0, NFULL, e_body, zeros)
                eqv = eqv + jnp.where(load_tail(r, NEG) == cur, 1.0, 0.0)
                ceq = jnp.sum(eqv)
                active = cnt < KTH + 1
                ans = jnp.where(active, cur, ans)
                thr = jnp.where(active, cur, thr)
                cnt = jnp.where(active, cnt + ceq, cnt)
                return (thr, cnt, ans)

            st = (jnp.float32(jnp.inf), jnp.float32(0.0), NEG)
            _, _, ans = lax.fori_loop(0, KTH + 1, extract, st)
            return ans

        tk = lax.cond(c_gt > KTH + 0.5, fallback, lambda: tk)

        gmx = group_max10(xbuf, r)
        gyv = group_max10(ybuf, r)
        gynv = group_max10(ynbuf, r)

        zvec = jnp.where(ii == L, tk, jnp.where(ii < L, gmx, 0.0))
        sigv = 1.0 / (1.0 + jnp.exp(-zvec))
        s_gm = jnp.where(ii < L, sigv, 0.0)
        sig_t = jnp.max(jnp.where(ii == L, sigv, NEG))
        thres = jnp.maximum(sig_t, 0.5)

        gt_m = gyv > 0.0
        neg_m = gynv > 0.0
        has_gt = jnp.any(gt_m)
        gsel = jnp.min(jnp.where(gt_m, ii, NL))
        gsel = jnp.where(has_gt, gsel, 0)

        nom = jnp.max(s_gm)
        inc_neg = jnp.max(jnp.where(neg_m, s_gm, 0.0))
        gt_max = jnp.max(jnp.where(ii == gsel, s_gm, 0.0))
        inc_max = jnp.max(jnp.where((ii != gsel) & (ii < L), s_gm, 0.0))

        d1 = nom - thres + ALPHA1
        d2 = inc_neg - thres + ALPHA1
        d3 = thres - gt_max + ALPHA1
        d4 = inc_max - thres + ALPHA1
        dvec = jnp.where(ii == 0, d1,
               jnp.where(ii == 1, d2,
               jnp.where(ii == 2, d3,
               jnp.where(ii == 3, d4, 0.0))))
        rv = jnp.where(dvec > 0, 2.0, 1.0) / (1.0 + jnp.exp(-ALPHA3 * dvec))

        c0 = jnp.where(has_gt, 0.0, 1.0 - ALPHA)
        c1 = jnp.where(has_gt, jnp.where(inc_neg > 0, ALPHA, 0.0), ALPHA)
        c2 = jnp.where(has_gt, 1.0, 0.0)
        c3 = jnp.where(has_gt,
                       jnp.where(inc_max > 0, 1.0 - ALPHA, 0.0)
                       + jnp.where(inc_neg > 0, 0.0, ALPHA), 0.0)
        cvec = jnp.where(ii == 0, c0,
               jnp.where(ii == 1, c1,
               jnp.where(ii == 2, c2,
               jnp.where(ii == 3, c3, 0.0))))
        loss_row = jnp.sum(cvec * rv)
        return acc + loss_row

    acc = lax.fori_loop(0, NSLAB, slab_body, jnp.float32(0.0))
    outb[...] = jnp.where(ii == 0, acc, 0.0)
    pltpu.sync_copy(outb, out_hbm.at[wid])


@jax.jit
def kernel(x, y, y_neg):
    mesh = plsc.VectorSubcoreMesh(core_axis_name="c", subcore_axis_name="s")
    run = functools.partial(
        pl.kernel,
        out_type=jax.ShapeDtypeStruct((NW, NL), jnp.float32),
        mesh=mesh,
        compiler_params=pltpu.CompilerParams(needs_layout_passes=False),
        scratch_types=[
            pltpu.VMEM((SLAB, C), jnp.float32),
            pltpu.VMEM((SLAB, L * GROUP), jnp.float32),
            pltpu.VMEM((SLAB, L * GROUP), jnp.float32),
            pltpu.VMEM((NL,), jnp.float32),
        ],
    )(_sc_body)
    partials = run(x, y, y_neg)
    return jnp.sum(partials[:, 0])


# speedup vs baseline: 19.1608x; 1.4892x over previous
import functools
import jax
import jax.numpy as jnp
from jax import lax
from jax.experimental import pallas as pl
from jax.experimental.pallas import tpu as pltpu
from jax.experimental.pallas import tpu_sc as plsc

B = 4096
C = 9605
L = 10
GROUP = 64
ALPHA = 0.3
ALPHA1 = 0.05
ALPHA3 = 5.0

NL = 16
NCHUNK = (C + NL - 1) // NL
NFULL = NCHUNK - 1
NTAIL = C - NFULL * NL
NW = 32
RPW = B // NW
NEG = float("-inf")
KTH = 10


def _sorted16(v, descending=False):
    k, _ = plsc.sort_key_val(v, v, descending=descending)
    return k


def _top4_merge(m1, m2, m3, m4):
    a = _sorted16(m1)
    b = _sorted16(m2, descending=True)
    h1 = _sorted16(jnp.maximum(a, b))
    c = _sorted16(m3)
    d = _sorted16(m4, descending=True)
    h2 = _sorted16(jnp.maximum(c, d), descending=True)
    hs = _sorted16(jnp.maximum(h1, h2))
    ii = lax.iota(jnp.int32, NL)
    return jnp.max(jnp.where(ii == NL - 1 - KTH, hs, NEG))


SLAB = 8
NSLAB = RPW // SLAB


def _sc_body(x_hbm, y_hbm, yn_hbm, out_hbm, xbuf, ybuf, ynbuf, outb):
    cid = lax.axis_index("c")
    sid = lax.axis_index("s")
    wid = sid * 2 + cid
    base = wid * RPW
    ii = lax.iota(jnp.int32, NL)
    zeros = jnp.zeros((NL,), jnp.float32)
    neginf = jnp.full((NL,), -jnp.inf, jnp.float32)
    tail_mask = ii < NTAIL

    def slab_body(sb, acc0):
        row0 = base + sb * SLAB
        pltpu.sync_copy(x_hbm.at[pl.ds(row0, SLAB), :], xbuf)
        pltpu.sync_copy(y_hbm.at[pl.ds(row0, SLAB), pl.ds(0, L * GROUP)], ybuf)
        pltpu.sync_copy(yn_hbm.at[pl.ds(row0, SLAB), pl.ds(0, L * GROUP)], ynbuf)
        return lax.fori_loop(0, SLAB, row_body, acc0)

    def load_tail(r, thr):
        idx_r = jnp.zeros((NL,), jnp.int32) + r
        idx_c = jnp.minimum(NFULL * NL + ii, C - 1)
        v = plsc.load_gather(xbuf, [idx_r, idx_c])
        return jnp.where(tail_mask, v, thr)

    def group_max10(buf, r):
        out = neginf
        for l in range(L):
            g0 = buf[r, pl.ds(l * GROUP, NL)]
            g1 = buf[r, pl.ds(l * GROUP + NL, NL)]
            g2 = buf[r, pl.ds(l * GROUP + 2 * NL, NL)]
            g3 = buf[r, pl.ds(l * GROUP + 3 * NL, NL)]
            s = jnp.max(jnp.maximum(jnp.maximum(g0, g1), jnp.maximum(g2, g3)))
            out = jnp.where(ii == l, s, out)
        return out

    def row_body(r, acc):
        def c_body(j, ms):
            m1, m2, m3, m4 = ms
            v = xbuf[r, pl.ds(j * NL, NL)]
            t = jnp.minimum(m1, v); m1 = jnp.maximum(m1, v)
            u = jnp.minimum(m2, t); m2 = jnp.maximum(m2, t)
            w = jnp.minimum(m3, u); m3 = jnp.maximum(m3, u)
            m4 = jnp.maximum(m4, w)
            return (m1, m2, m3, m4)

        m1, m2, m3, m4 = lax.fori_loop(
            0, NFULL, c_body, (neginf, neginf, neginf, neginf), unroll=8)
        vl = load_tail(r, neginf)
        t = jnp.minimum(m1, vl); m1 = jnp.maximum(m1, vl)
        u = jnp.minimum(m2, t); m2 = jnp.maximum(m2, t)
        w = jnp.minimum(m3, u); m3 = jnp.maximum(m3, u)
        m4 = jnp.maximum(m4, w)

        tk = _top4_merge(m1, m2, m3, m4)

        def v_body(j, cnt):
            v = xbuf[r, pl.ds(j * NL, NL)]
            return cnt + jnp.where(v > tk, 1.0, 0.0)

        cntv = lax.fori_loop(0, NFULL, v_body, zeros, unroll=8)
        cntv = cntv + jnp.where(load_tail(r, NEG) > tk, 1.0, 0.0)
        c_gt = jnp.sum(cntv)

        def fallback():
            def extract(_, st):
                thr, cnt, ans = st

                def p_body(j, mc):
                    mx, ct = mc
                    v = xbuf[r, pl.ds(j * NL, NL)]
                    below = jnp.where(v < thr, v, -jnp.inf)
                    mx = jnp.maximum(mx, below)
                    return (mx, ct)

                mx, _ = lax.fori_loop(0, NFULL, p_body, (neginf, zeros))
                vt = load_tail(r, thr)
                mx = jnp.maximum(mx, jnp.where(vt < thr, vt, -jnp.inf))
                cur = jnp.max(mx)

                def e_body(j, ct):
                    v = xbuf[r, pl.ds(j * NL, NL)]
                    return ct + jnp.where(v == cur, 1.0, 0.0)

                eqv = lax.fori_loop(0, NFULL, e_body, zeros)
                eqv = eqv + jnp.where(load_tail(r, NEG) == cur, 1.0, 0.0)
                ceq = jnp.sum(eqv)
                active = cnt < KTH + 1
                ans = jnp.where(active, cur, ans)
                thr = jnp.where(active, cur, thr)
                cnt = jnp.where(active, cnt + ceq, cnt)
                return (thr, cnt, ans)

            st = (jnp.float32(jnp.inf), jnp.float32(0.0), NEG)
            _, _, ans = lax.fori_loop(0, KTH + 1, extract, st)
            return ans

        tk = lax.cond(c_gt > KTH + 0.5, fallback, lambda: tk)

        gmx = group_max10(xbuf, r)
        gyv = group_max10(ybuf, r)
        gynv = group_max10(ynbuf, r)

        zvec = jnp.where(ii == L, tk, jnp.where(ii < L, gmx, 0.0))
        sigv = 1.0 / (1.0 + jnp.exp(-zvec))
        s_gm = jnp.where(ii < L, sigv, 0.0)
        sig_t = jnp.max(jnp.where(ii == L, sigv, NEG))
        thres = jnp.maximum(sig_t, 0.5)

        gt_m = gyv > 0.0
        neg_m = gynv > 0.0
        has_gt = jnp.any(gt_m)
        gsel = jnp.min(jnp.where(gt_m, ii, NL))
        gsel = jnp.where(has_gt, gsel, 0)

        nom = jnp.max(s_gm)
        inc_neg = jnp.max(jnp.where(neg_m, s_gm, 0.0))
        gt_max = jnp.max(jnp.where(ii == gsel, s_gm, 0.0))
        inc_max = jnp.max(jnp.where((ii != gsel) & (ii < L), s_gm, 0.0))

        d1 = nom - thres + ALPHA1
        d2 = inc_neg - thres + ALPHA1
        d3 = thres - gt_max + ALPHA1
        d4 = inc_max - thres + ALPHA1
        dvec = jnp.where(ii == 0, d1,
               jnp.where(ii == 1, d2,
               jnp.where(ii == 2, d3,
               jnp.where(ii == 3, d4, 0.0))))
        rv = jnp.where(dvec > 0, 2.0, 1.0) / (1.0 + jnp.exp(-ALPHA3 * dvec))

        c0 = jnp.where(has_gt, 0.0, 1.0 - ALPHA)
        c1 = jnp.where(has_gt, jnp.where(inc_neg > 0, ALPHA, 0.0), ALPHA)
        c2 = jnp.where(has_gt, 1.0, 0.0)
        c3 = jnp.where(has_gt,
                       jnp.where(inc_max > 0, 1.0 - ALPHA, 0.0)
                       + jnp.where(inc_neg > 0, 0.0, ALPHA), 0.0)
        cvec = jnp.where(ii == 0, c0,
               jnp.where(ii == 1, c1,
               jnp.where(ii == 2, c2,
               jnp.where(ii == 3, c3, 0.0))))
        loss_row = jnp.sum(cvec * rv)
        return acc + loss_row

    acc = lax.fori_loop(0, NSLAB, slab_body, jnp.float32(0.0))
    outb[...] = jnp.where(ii == 0, acc, 0.0)
    pltpu.sync_copy(outb, out_hbm.at[wid])


@jax.jit
def kernel(x, y, y_neg):
    mesh = plsc.VectorSubcoreMesh(core_axis_name="c", subcore_axis_name="s")
    run = functools.partial(
        pl.kernel,
        out_type=jax.ShapeDtypeStruct((NW, NL), jnp.float32),
        mesh=mesh,
        compiler_params=pltpu.CompilerParams(needs_layout_passes=False),
        scratch_types=[
            pltpu.VMEM((SLAB, C), jnp.float32),
            pltpu.VMEM((SLAB, L * GROUP), jnp.float32),
            pltpu.VMEM((SLAB, L * GROUP), jnp.float32),
            pltpu.VMEM((NL,), jnp.float32),
        ],
    )(_sc_body)
    partials = run(x, y, y_neg)
    return jnp.sum(partials[:, 0])


# speedup vs baseline: 21.3076x; 1.1120x over previous
import functools
import jax
import jax.numpy as jnp
from jax import lax
from jax.experimental import pallas as pl
from jax.experimental.pallas import tpu as pltpu
from jax.experimental.pallas import tpu_sc as plsc

B = 4096
C = 9605
L = 10
GROUP = 64
ALPHA = 0.3
ALPHA1 = 0.05
ALPHA3 = 5.0

NL = 16
NCHUNK = (C + NL - 1) // NL
NFULL = NCHUNK - 1
NTAIL = C - NFULL * NL
NW = 32
RPW = B // NW
NEG = float("-inf")
KTH = 10


def _sorted16(v, descending=False):
    k, _ = plsc.sort_key_val(v, v, descending=descending)
    return k


def _top4_merge(m1, m2, m3, m4):
    a = _sorted16(m1)
    b = _sorted16(m2, descending=True)
    h1 = _sorted16(jnp.maximum(a, b))
    c = _sorted16(m3)
    d = _sorted16(m4, descending=True)
    h2 = _sorted16(jnp.maximum(c, d), descending=True)
    hs = _sorted16(jnp.maximum(h1, h2))
    ii = lax.iota(jnp.int32, NL)
    return jnp.max(jnp.where(ii == NL - 1 - KTH, hs, NEG))


SLAB = 8
NSLAB = RPW // SLAB


def _sc_body(x_hbm, y_hbm, yn_hbm, out_hbm, xbuf, ybuf, ynbuf, outb):
    cid = lax.axis_index("c")
    sid = lax.axis_index("s")
    wid = sid * 2 + cid
    base = wid * RPW
    ii = lax.iota(jnp.int32, NL)
    zeros = jnp.zeros((NL,), jnp.float32)
    neginf = jnp.full((NL,), -jnp.inf, jnp.float32)
    tail_mask = ii < NTAIL

    def slab_body(sb, acc0):
        row0 = base + sb * SLAB
        pltpu.sync_copy(x_hbm.at[pl.ds(row0, SLAB), :], xbuf)
        pltpu.sync_copy(y_hbm.at[pl.ds(row0, SLAB), pl.ds(0, L * GROUP)], ybuf)
        pltpu.sync_copy(yn_hbm.at[pl.ds(row0, SLAB), pl.ds(0, L * GROUP)], ynbuf)
        return lax.fori_loop(0, SLAB, row_body, acc0)

    def load_tail(r, thr):
        idx_r = jnp.zeros((NL,), jnp.int32) + r
        idx_c = jnp.minimum(NFULL * NL + ii, C - 1)
        v = plsc.load_gather(xbuf, [idx_r, idx_c])
        return jnp.where(tail_mask, v, thr)

    def group_max10(buf, r):
        out = neginf
        for l in range(L):
            g0 = buf[r, pl.ds(l * GROUP, NL)]
            g1 = buf[r, pl.ds(l * GROUP + NL, NL)]
            g2 = buf[r, pl.ds(l * GROUP + 2 * NL, NL)]
            g3 = buf[r, pl.ds(l * GROUP + 3 * NL, NL)]
            s = jnp.max(jnp.maximum(jnp.maximum(g0, g1), jnp.maximum(g2, g3)))
            out = jnp.where(ii == l, s, out)
        return out

    def row_body(r, acc):
        def c_body(j, ms):
            m1, m2, m3, m4 = ms
            v = xbuf[r, pl.ds(j * NL, NL)]
            t = jnp.minimum(m1, v); m1 = jnp.maximum(m1, v)
            u = jnp.minimum(m2, t); m2 = jnp.maximum(m2, t)
            w = jnp.minimum(m3, u); m3 = jnp.maximum(m3, u)
            m4 = jnp.maximum(m4, w)
            return (m1, m2, m3, m4)

        m1, m2, m3, m4 = lax.fori_loop(
            0, NFULL, c_body, (neginf, neginf, neginf, neginf), unroll=8)
        vl = load_tail(r, neginf)
        t = jnp.minimum(m1, vl); m1 = jnp.maximum(m1, vl)
        u = jnp.minimum(m2, t); m2 = jnp.maximum(m2, t)
        w = jnp.minimum(m3, u); m3 = jnp.maximum(m3, u)
        m4 = jnp.maximum(m4, w)

        tk = _top4_merge(m1, m2, m3, m4)

        hid = jnp.max(m4) > tk

        def fallback():
            def extract(_, st):
                thr, cnt, ans = st

                def p_body(j, mc):
                    mx, ct = mc
                    v = xbuf[r, pl.ds(j * NL, NL)]
                    below = jnp.where(v < thr, v, -jnp.inf)
                    mx = jnp.maximum(mx, below)
                    return (mx, ct)

                mx, _ = lax.fori_loop(0, NFULL, p_body, (neginf, zeros))
                vt = load_tail(r, thr)
                mx = jnp.maximum(mx, jnp.where(vt < thr, vt, -jnp.inf))
                cur = jnp.max(mx)

                def e_body(j, ct):
                    v = xbuf[r, pl.ds(j * NL, NL)]
                    return ct + jnp.where(v == cur, 1.0, 0.0)

                eqv = lax.fori_loop(0, NFULL, e_body, zeros)
                eqv = eqv + jnp.where(load_tail(r, NEG) == cur, 1.0, 0.0)
                ceq = jnp.sum(eqv)
                active = cnt < KTH + 1
                ans = jnp.where(active, cur, ans)
                thr = jnp.where(active, cur, thr)
                cnt = jnp.where(active, cnt + ceq, cnt)
                return (thr, cnt, ans)

            st = (jnp.float32(jnp.inf), jnp.float32(0.0), NEG)
            _, _, ans = lax.fori_loop(0, KTH + 1, extract, st)
            return ans

        def verify():
            def v_body(j, cnt):
                v = xbuf[r, pl.ds(j * NL, NL)]
                return cnt + jnp.where(v > tk, 1.0, 0.0)

            cntv = lax.fori_loop(0, NFULL, v_body, zeros, unroll=8)
            cntv = cntv + jnp.where(load_tail(r, NEG) > tk, 1.0, 0.0)
            c_gt = jnp.sum(cntv)
            return lax.cond(c_gt > KTH + 0.5, fallback, lambda: tk)

        tk = lax.cond(hid, verify, lambda: tk)

        gmx = group_max10(xbuf, r)
        gyv = group_max10(ybuf, r)
        gynv = group_max10(ynbuf, r)

        zvec = jnp.where(ii == L, tk, jnp.where(ii < L, gmx, 0.0))
        sigv = 1.0 / (1.0 + jnp.exp(-zvec))
        s_gm = jnp.where(ii < L, sigv, 0.0)
        sig_t = jnp.max(jnp.where(ii == L, sigv, NEG))
        thres = jnp.maximum(sig_t, 0.5)

        gt_m = gyv > 0.0
        neg_m = gynv > 0.0
        has_gt = jnp.any(gt_m)
        gsel = jnp.min(jnp.where(gt_m, ii, NL))
        gsel = jnp.where(has_gt, gsel, 0)

        nom = jnp.max(s_gm)
        inc_neg = jnp.max(jnp.where(neg_m, s_gm, 0.0))
        gt_max = jnp.max(jnp.where(ii == gsel, s_gm, 0.0))
        inc_max = jnp.max(jnp.where((ii != gsel) & (ii < L), s_gm, 0.0))

        d1 = nom - thres + ALPHA1
        d2 = inc_neg - thres + ALPHA1
        d3 = thres - gt_max + ALPHA1
        d4 = inc_max - thres + ALPHA1
        dvec = jnp.where(ii == 0, d1,
               jnp.where(ii == 1, d2,
               jnp.where(ii == 2, d3,
               jnp.where(ii == 3, d4, 0.0))))
        rv = jnp.where(dvec > 0, 2.0, 1.0) / (1.0 + jnp.exp(-ALPHA3 * dvec))

        c0 = jnp.where(has_gt, 0.0, 1.0 - ALPHA)
        c1 = jnp.where(has_gt, jnp.where(inc_neg > 0, ALPHA, 0.0), ALPHA)
        c2 = jnp.where(has_gt, 1.0, 0.0)
        c3 = jnp.where(has_gt,
                       jnp.where(inc_max > 0, 1.0 - ALPHA, 0.0)
                       + jnp.where(inc_neg > 0, 0.0, ALPHA), 0.0)
        cvec = jnp.where(ii == 0, c0,
               jnp.where(ii == 1, c1,
               jnp.where(ii == 2, c2,
               jnp.where(ii == 3, c3, 0.0))))
        loss_row = jnp.sum(cvec * rv)
        return acc + loss_row

    acc = lax.fori_loop(0, NSLAB, slab_body, jnp.float32(0.0))
    outb[...] = jnp.where(ii == 0, acc, 0.0)
    pltpu.sync_copy(outb, out_hbm.at[wid])


@jax.jit
def kernel(x, y, y_neg):
    mesh = plsc.VectorSubcoreMesh(core_axis_name="c", subcore_axis_name="s")
    run = functools.partial(
        pl.kernel,
        out_type=jax.ShapeDtypeStruct((NW, NL), jnp.float32),
        mesh=mesh,
        compiler_params=pltpu.CompilerParams(needs_layout_passes=False),
        scratch_types=[
            pltpu.VMEM((SLAB, C), jnp.float32),
            pltpu.VMEM((SLAB, L * GROUP), jnp.float32),
            pltpu.VMEM((SLAB, L * GROUP), jnp.float32),
            pltpu.VMEM((NL,), jnp.float32),
        ],
    )(_sc_body)
    partials = run(x, y, y_neg)
    return jnp.sum(partials[:, 0])
